# baseline (speedup 1.0000x reference)
import functools

import jax
import jax.numpy as jnp
from jax import lax
from jax.experimental import pallas as pl
from jax.experimental.pallas import tpu as pltpu
from jax.experimental.pallas import tpu_sc as plsc

_NUM_CORES = 2
_NUM_SUBCORES = 16
_NUM_WORKERS = _NUM_CORES * _NUM_SUBCORES

_CHUNK = 8
_NBUF = 13
_DEPTH = 6


def _emb_lookup(src, tgt, seg, table):
    b, s = src.shape
    _, hidden = table.shape
    n_per_w = (b * s) // _NUM_WORKERS
    steps = n_per_w // _CHUNK
    w_per_row = s // n_per_w
    assert (steps - 2 * _DEPTH) % _NBUF == 0 and _NBUF >= 2 * _DEPTH
    mesh = plsc.VectorSubcoreMesh(core_axis_name="c", subcore_axis_name="s")

    @functools.partial(
        pl.kernel,
        out_type=(
            jax.ShapeDtypeStruct((b, s, hidden), jnp.float32),
            jax.ShapeDtypeStruct(tgt.shape, tgt.dtype),
            jax.ShapeDtypeStruct(seg.shape, seg.dtype),
        ),
        mesh=mesh,
        scratch_types=[
            pltpu.VMEM((n_per_w,), jnp.int32),
            pltpu.VMEM((n_per_w,), jnp.int32),
            pltpu.VMEM((n_per_w,), jnp.int32),
            pltpu.VMEM((_NBUF, _CHUNK, hidden), jnp.float32),
            pltpu.SemaphoreType.DMA,
        ]
        + [pltpu.SemaphoreType.DMA] * (2 * _NBUF),
    )
    def emb(idx_hbm, tgt_hbm, seg_hbm, table_hbm, out_hbm, tgt_out, seg_out,
            idx_v, tgt_v, seg_v, rows, xsem, *sems):
        gsem = sems[:_NBUF]
        psem = sems[_NBUF:]
        wid = lax.axis_index("s") * _NUM_CORES + lax.axis_index("c")
        row = wid // w_per_row
        col = (wid % w_per_row) * n_per_w

        pltpu.sync_copy(idx_hbm.at[row, pl.ds(col, n_per_w)], idx_v)
        pltpu.async_copy(tgt_hbm.at[row, pl.ds(col, n_per_w)], tgt_v, xsem)
        pltpu.async_copy(seg_hbm.at[row, pl.ds(col, n_per_w)], seg_v, xsem)

        def start_gather(st, k):
            pltpu.async_copy(
                table_hbm.at[idx_v.at[pl.ds(st * _CHUNK, _CHUNK)]],
                rows.at[k],
                gsem[k],
            )

        def start_put(st, k):
            pltpu.async_copy(
                rows.at[k], out_hbm.at[row, pl.ds(col + st * _CHUNK, _CHUNK)],
                psem[k],
            )

        def wait_gather(k):
            pltpu.make_async_copy(
                table_hbm.at[pl.ds(0, _CHUNK)], rows.at[k], gsem[k]
            ).wait()

        def wait_put(k):
            pltpu.make_async_copy(
                rows.at[k], out_hbm.at[0, pl.ds(0, _CHUNK)], psem[k]
            ).wait()

        for t in range(_DEPTH):
            start_gather(t, t)
        for st in range(_DEPTH):
            wait_gather(st)
            start_put(st, st)
            start_gather(st + _DEPTH, st + _DEPTH)

        pltpu.make_async_copy(tgt_hbm.at[0, pl.ds(0, n_per_w)], tgt_v, xsem).wait()
        pltpu.make_async_copy(seg_hbm.at[0, pl.ds(0, n_per_w)], seg_v, xsem).wait()
        pltpu.async_copy(tgt_v, tgt_out.at[row, pl.ds(col, n_per_w)], xsem)
        pltpu.async_copy(seg_v, seg_out.at[row, pl.ds(col, n_per_w)], xsem)

        putted = set(range(_DEPTH))

        def step_body(gr_base, k, skip_ok):
            st = gr_base + k + _DEPTH
            wait_gather((k + _DEPTH) % _NBUF)
            start_put(st, (k + _DEPTH) % _NBUF)
            nb = (k + 2 * _DEPTH) % _NBUF
            if not (skip_ok and nb not in putted):
                wait_put(nb)
            start_gather(st + _DEPTH, nb)

        for k in range(_NBUF):
            step_body(0, k, True)
            putted.add((k + _DEPTH) % _NBUF)

        def group(gr, carry):
            for k in range(_NBUF):
                step_body(_NBUF * gr, k, False)
            return carry

        lax.fori_loop(1, (steps - 2 * _DEPTH) // _NBUF, group, 0)

        for st in range(steps - _DEPTH, steps):
            wait_gather(st % _NBUF)
            start_put(st, st % _NBUF)
        for k in range(_NBUF):
            wait_put(k)
        pltpu.make_async_copy(tgt_v, tgt_out.at[0, pl.ds(0, n_per_w)], xsem).wait()
        pltpu.make_async_copy(seg_v, seg_out.at[0, pl.ds(0, n_per_w)], xsem).wait()

    return emb(src, tgt, seg, table)


def kernel(src, tgt, seg, word_table):
    return _emb_lookup(src.astype(jnp.int32), tgt, seg, word_table)


# speedup vs baseline: 1.0070x; 1.0070x over previous
import functools

import jax
import jax.numpy as jnp
from jax import lax
from jax.experimental import pallas as pl
from jax.experimental.pallas import tpu as pltpu
from jax.experimental.pallas import tpu_sc as plsc

_NUM_CORES = 2
_NUM_SUBCORES = 16
_NUM_WORKERS = _NUM_CORES * _NUM_SUBCORES

_CHUNK = 8
_NBUF = 8
_DEPTH = _NBUF // 2


def _emb_lookup(src, tgt, seg, table):
    b, s = src.shape
    _, hidden = table.shape
    n_per_w = (b * s) // _NUM_WORKERS
    steps = n_per_w // _CHUNK
    w_per_row = s // n_per_w
    assert steps % _NBUF == 0 and steps >= 2 * _NBUF
    mesh = plsc.VectorSubcoreMesh(core_axis_name="c", subcore_axis_name="s")

    @functools.partial(
        pl.kernel,
        out_type=(
            jax.ShapeDtypeStruct((b, s, hidden), jnp.float32),
            jax.ShapeDtypeStruct(tgt.shape, tgt.dtype),
            jax.ShapeDtypeStruct(seg.shape, seg.dtype),
        ),
        mesh=mesh,
        scratch_types=[
            pltpu.VMEM((n_per_w,), jnp.int32),
            pltpu.VMEM((n_per_w,), jnp.int32),
            pltpu.VMEM((n_per_w,), jnp.int32),
            pltpu.VMEM((_NBUF, _CHUNK, hidden), jnp.float32),
            pltpu.SemaphoreType.DMA,
        ]
        + [pltpu.SemaphoreType.DMA] * (2 * _NBUF),
    )
    def emb(idx_hbm, tgt_hbm, seg_hbm, table_hbm, out_hbm, tgt_out, seg_out,
            idx_v, tgt_v, seg_v, rows, xsem, *sems):
        gsem = sems[:_NBUF]
        psem = sems[_NBUF:]
        wid = lax.axis_index("s") * _NUM_CORES + lax.axis_index("c")
        row = wid // w_per_row
        col = (wid % w_per_row) * n_per_w

        pltpu.sync_copy(idx_hbm.at[row, pl.ds(col, n_per_w)], idx_v)
        pltpu.async_copy(tgt_hbm.at[row, pl.ds(col, n_per_w)], tgt_v, xsem)
        pltpu.async_copy(seg_hbm.at[row, pl.ds(col, n_per_w)], seg_v, xsem)

        def start_gather(st, k):
            pltpu.async_copy(
                table_hbm.at[idx_v.at[pl.ds(st * _CHUNK, _CHUNK)]],
                rows.at[k],
                gsem[k],
            )

        def start_put(st, k):
            pltpu.async_copy(
                rows.at[k], out_hbm.at[row, pl.ds(col + st * _CHUNK, _CHUNK)],
                psem[k],
            )

        def wait_gather(k):
            pltpu.make_async_copy(
                table_hbm.at[pl.ds(0, _CHUNK)], rows.at[k], gsem[k]
            ).wait()

        def wait_put(k):
            pltpu.make_async_copy(
                rows.at[k], out_hbm.at[0, pl.ds(0, _CHUNK)], psem[k]
            ).wait()

        for t in range(_DEPTH):
            start_gather(t, t)
        for st in range(_DEPTH):
            wait_gather(st)
            start_put(st, st)
            start_gather(st + _DEPTH, st + _DEPTH)

        pltpu.make_async_copy(tgt_hbm.at[0, pl.ds(0, n_per_w)], tgt_v, xsem).wait()
        pltpu.make_async_copy(seg_hbm.at[0, pl.ds(0, n_per_w)], seg_v, xsem).wait()
        pltpu.async_copy(tgt_v, tgt_out.at[row, pl.ds(col, n_per_w)], xsem)
        pltpu.async_copy(seg_v, seg_out.at[row, pl.ds(col, n_per_w)], xsem)

        def group(gr, carry):
            for k in range(_NBUF):
                st = _NBUF * gr + k + _DEPTH
                wait_gather((k + _DEPTH) % _NBUF)
                start_put(st, (k + _DEPTH) % _NBUF)
                wait_put(k % _NBUF)
                start_gather(st + _DEPTH, k % _NBUF)
            return carry

        lax.fori_loop(0, (steps - 2 * _DEPTH) // _NBUF, group, 0)

        for st in range(steps - _DEPTH, steps):
            wait_gather(st % _NBUF)
            start_put(st, st % _NBUF)
        for k in range(_NBUF):
            wait_put(k)
        pltpu.make_async_copy(tgt_v, tgt_out.at[0, pl.ds(0, n_per_w)], xsem).wait()
        pltpu.make_async_copy(seg_v, seg_out.at[0, pl.ds(0, n_per_w)], xsem).wait()

    return emb(src, tgt, seg, table)


def kernel(src, tgt, seg, word_table):
    return _emb_lookup(src.astype(jnp.int32), tgt, seg, word_table)
